# Initial kernel scaffold; baseline (speedup 1.0000x reference)
import jax
import jax.numpy as jnp
from jax.experimental import pallas as pl


def kernel(x, edge_index, batch, fe_w, fe_b, wq, bq, wk, bk, wv, bv, ws, bs):
    raise NotImplementedError("write your pallas kernel here")


# speedup vs baseline: 2.3990x; 2.3990x over previous
import jax
import jax.numpy as jnp
from jax import lax
from jax.experimental import pallas as pl
from jax.experimental.pallas import tpu as pltpu
from jax.experimental.pallas import tpu_sc as plsc

_N = 10000
_E = 320000
_D = 128
_G = 16
_DEPTH = 4
_H = 64
_VW = 80
_C = 128
_NCHUNK = _E // _C
_NC = 2
_NS = 16
_NW = _NC * _NS
_CPW = (_NCHUNK + _NW - 1) // _NW
_RPT = _N // _NS
_SCALE = 1.0 / float(_D) ** 0.5


def _proj(h, wq, bq, wk, bk, wv1, wv2, bv, ws, bs,
          q_ref, k_ref, v1_ref, v2_ref, s_ref):
    q_ref[...] = jnp.dot(h, wq, preferred_element_type=jnp.float32) + bq
    k_ref[...] = jnp.dot(h, wk, preferred_element_type=jnp.float32) + bk
    v1 = jnp.dot(h, wv1, preferred_element_type=jnp.float32) + bv[:, :_H]
    v2 = jnp.dot(h, wv2, preferred_element_type=jnp.float32) + bv[:, _H:]
    lane = lax.broadcasted_iota(jnp.int32, (_N, _VW - _H), 1)
    ones_col = jnp.where(lane == 0, 1.0, 0.0).astype(jnp.float32)
    v1_ref[...] = jnp.concatenate([v1, ones_col], axis=1)
    v2_ref[...] = jnp.concatenate([v2, jnp.zeros_like(ones_col)], axis=1)
    s_ref[...] = jnp.dot(h, ws, preferred_element_type=jnp.float32) + bs


def _encode_proj_body(x_ref, few_ref, feb_ref, wq_ref, bq_ref, wk_ref, bk_ref,
                      wv1_ref, wv2_ref, bv_ref, ws_ref, bs_ref,
                      q_ref, k_ref, v1_ref, v2_ref, s_ref):
    x = x_ref[...]
    xa = jnp.dot(x, few_ref[...], preferred_element_type=jnp.float32)
    xa = xa + feb_ref[...]
    h = jnp.concatenate([x, xa], axis=1)
    _proj(h, wq_ref[...], bq_ref[...], wk_ref[...], bk_ref[...],
          wv1_ref[...], wv2_ref[...], bv_ref[...], ws_ref[...], bs_ref[...],
          q_ref, k_ref, v1_ref, v2_ref, s_ref)


def _combine_pool(agg_ref, skip_ref, b_ref, pooled_ref):
    agg_a = agg_ref[0] + agg_ref[1]
    agg_b = agg_ref[2] + agg_ref[3]
    den = agg_a[:, _H:_H + 1] + 1e-16
    num = jnp.concatenate([agg_a[:, :_H], agg_b[:, :_H]], axis=1)
    h = num / den + skip_ref[...]
    g_iota = lax.broadcasted_iota(jnp.int32, (_G, _N), 0)
    onehot = (g_iota == b_ref[...]).astype(jnp.float32)
    cnt = jnp.maximum(jnp.sum(onehot, axis=1, keepdims=True), 1.0)
    pooled_ref[...] = (
        jnp.dot(onehot, h, preferred_element_type=jnp.float32) / cnt)
    return h


def _combine_proj_body(agg_ref, skip_ref, b_ref, wq_ref, bq_ref, wk_ref,
                       bk_ref, wv1_ref, wv2_ref, bv_ref, ws_ref, bs_ref,
                       pooled_ref, q_ref, k_ref, v1_ref, v2_ref, s_ref):
    h = _combine_pool(agg_ref, skip_ref, b_ref, pooled_ref)
    _proj(h, wq_ref[...], bq_ref[...], wk_ref[...], bk_ref[...],
          wv1_ref[...], wv2_ref[...], bv_ref[...], ws_ref[...], bs_ref[...],
          q_ref, k_ref, v1_ref, v2_ref, s_ref)


def _combine_final_body(agg_ref, skip_ref, b_ref, pooled_ref):
    _combine_pool(agg_ref, skip_ref, b_ref, pooled_ref)


_f32 = jnp.float32
_nodes = jax.ShapeDtypeStruct((_N, _D), _f32)
_vhalf = jax.ShapeDtypeStruct((_N, _VW), _f32)
_pooled = jax.ShapeDtypeStruct((_G, _D), _f32)

_encode_proj = pl.pallas_call(
    _encode_proj_body, out_shape=[_nodes, _nodes, _vhalf, _vhalf, _nodes])

_combine_proj = pl.pallas_call(
    _combine_proj_body,
    out_shape=[_pooled, _nodes, _nodes, _vhalf, _vhalf, _nodes])

_combine_final = pl.pallas_call(_combine_final_body, out_shape=[_pooled])


def _zero_acc(zero_hbm, acc_sh, sid):
    pltpu.sync_copy(zero_hbm.at[pl.ds(sid * _RPT, _RPT)],
                    acc_sh.at[pl.ds(sid * _RPT, _RPT)])


def _sc_edge_body(q_hbm, k_hbm, v1_hbm, v2_hbm, src_hbm, dst_hbm, zero_hbm,
                  out_hbm,
                  sidx, didx, q_rows, k_rows, v_rows, alpha_all, acc_sh):
    cid = lax.axis_index("c")
    sid = lax.axis_index("s")
    wid = cid * _NS + sid
    lane = lax.iota(jnp.int32, 16)

    _zero_acc(zero_hbm, acc_sh, sid)
    plsc.subcore_barrier()

    @pl.loop(0, _CPW)
    def _pass_a(i):
        chunk = wid + i * _NW

        @pl.when(chunk < _NCHUNK)
        def _():
            pltpu.sync_copy(src_hbm.at[pl.ds(chunk, 1)], sidx)
            pltpu.sync_copy(dst_hbm.at[pl.ds(chunk, 1)], didx)
            pltpu.sync_copy(q_hbm.at[didx.at[0]], q_rows)
            pltpu.sync_copy(k_hbm.at[sidx.at[0]], k_rows)
            pltpu.sync_copy(v1_hbm.at[sidx.at[0]], v_rows)

            for g in range(_C // 16):
                rows = g * 16 + lane

                @pl.loop(0, _D, init_carry=jnp.zeros((16,), _f32), unroll=8)
                def _dot(d, acc):
                    colv = jnp.full((16,), d, jnp.int32)
                    qv = plsc.load_gather(q_rows, [rows, colv])
                    kv = plsc.load_gather(k_rows, [rows, colv])
                    return acc + qv * kv

                alpha = jnp.exp(_dot * _SCALE)
                alpha_all[pl.ds(i * _C + g * 16, 16)] = alpha

                @pl.loop(0, _VW, unroll=8)
                def _vscale(d):
                    colv = jnp.full((16,), d, jnp.int32)
                    vv = plsc.load_gather(v_rows, [rows, colv])
                    plsc.store_scatter(v_rows, [rows, colv], vv * alpha)

            pltpu.sync_copy(v_rows, acc_sh.at[didx.at[0]], add=True)

    plsc.subcore_barrier()
    pltpu.sync_copy(acc_sh.at[pl.ds(sid * _RPT, _RPT)],
                    out_hbm.at[cid, pl.ds(sid * _RPT, _RPT)])
    _zero_acc(zero_hbm, acc_sh, sid)
    plsc.subcore_barrier()

    @pl.loop(0, _CPW)
    def _pass_b(i):
        chunk = wid + i * _NW

        @pl.when(chunk < _NCHUNK)
        def _():
            pltpu.sync_copy(src_hbm.at[pl.ds(chunk, 1)], sidx)
            pltpu.sync_copy(dst_hbm.at[pl.ds(chunk, 1)], didx)
            pltpu.sync_copy(v2_hbm.at[sidx.at[0]], v_rows)

            for g in range(_C // 16):
                rows = g * 16 + lane
                alpha = alpha_all[pl.ds(i * _C + g * 16, 16)]

                @pl.loop(0, _VW, unroll=8)
                def _vscale(d):
                    colv = jnp.full((16,), d, jnp.int32)
                    vv = plsc.load_gather(v_rows, [rows, colv])
                    plsc.store_scatter(v_rows, [rows, colv], vv * alpha)

            pltpu.sync_copy(v_rows, acc_sh.at[didx.at[0]], add=True)

    plsc.subcore_barrier()
    pltpu.sync_copy(acc_sh.at[pl.ds(sid * _RPT, _RPT)],
                    out_hbm.at[_NC + cid, pl.ds(sid * _RPT, _RPT)])


def _sc_edge(q, k, v1, v2, src, dst, zeros):
    mesh = plsc.VectorSubcoreMesh(core_axis_name="c", subcore_axis_name="s")
    fn = pl.kernel(
        _sc_edge_body,
        out_type=jax.ShapeDtypeStruct((2 * _NC, _N, _VW), _f32),
        mesh=mesh,
        compiler_params=pltpu.CompilerParams(
            use_tc_tiling_on_sc=False, needs_layout_passes=False),
        scratch_types=[
            pltpu.VMEM((1, _C), jnp.int32),
            pltpu.VMEM((1, _C), jnp.int32),
            pltpu.VMEM((_C, _D), _f32),
            pltpu.VMEM((_C, _D), _f32),
            pltpu.VMEM((_C, _VW), _f32),
            pltpu.VMEM((_CPW * _C,), _f32),
            pltpu.VMEM_SHARED((_N, _VW), _f32),
        ],
    )
    return fn(q, k, v1, v2, src, dst, zeros)


def kernel(x, edge_index, batch, fe_w, fe_b, wq, bq, wk, bk, wv, bv, ws, bs):
    src = edge_index[0].reshape(_NCHUNK, _C)
    dst = edge_index[1].reshape(_NCHUNK, _C)
    batch_row = batch.reshape(1, _N)
    zeros = jnp.zeros((_N, _VW), _f32)

    q, k, v1, v2, s = _encode_proj(
        x, fe_w, fe_b[None], wq[0], bq[0][None], wk[0], bk[0][None],
        wv[0][:, :_H], wv[0][:, _H:], bv[0][None], ws[0], bs[0][None])

    encs = []
    for l in range(_DEPTH):
        agg = _sc_edge(q, k, v1, v2, src, dst, zeros)
        if l + 1 < _DEPTH:
            pooled, q, k, v1, v2, s = _combine_proj(
                agg, s, batch_row, wq[l + 1], bq[l + 1][None], wk[l + 1],
                bk[l + 1][None], wv[l + 1][:, :_H], wv[l + 1][:, _H:],
                bv[l + 1][None], ws[l + 1], bs[l + 1][None])
        else:
            (pooled,) = _combine_final(agg, s, batch_row)
        encs.append(pooled)
    return jnp.concatenate(encs, axis=-1)


# speedup vs baseline: 2.9561x; 1.2322x over previous
import jax
import jax.numpy as jnp
from jax import lax
from jax.experimental import pallas as pl
from jax.experimental.pallas import tpu as pltpu
from jax.experimental.pallas import tpu_sc as plsc

_N = 10000
_E = 320000
_D = 128
_G = 16
_DEPTH = 4
_H = 64
_VW = 80
_C = 64
_NCHUNK = _E // _C
_NC = 2
_NS = 16
_NW = _NC * _NS
_CPW = (_NCHUNK + _NW - 1) // _NW
_RPT = _N // _NS
_SCALE = 1.0 / float(_D) ** 0.5


def _proj(h, wq, bq, wk, bk, wv1, wv2, bv, ws, bs,
          q_ref, k_ref, v1_ref, v2_ref, s_ref):
    q_ref[...] = jnp.dot(h, wq, preferred_element_type=jnp.float32) + bq
    k_ref[...] = jnp.dot(h, wk, preferred_element_type=jnp.float32) + bk
    v1 = jnp.dot(h, wv1, preferred_element_type=jnp.float32) + bv[:, :_H]
    v2 = jnp.dot(h, wv2, preferred_element_type=jnp.float32) + bv[:, _H:]
    lane = lax.broadcasted_iota(jnp.int32, (_N, _VW - _H), 1)
    ones_col = jnp.where(lane == 0, 1.0, 0.0).astype(jnp.float32)
    v1_ref[...] = jnp.concatenate([v1, ones_col], axis=1)
    v2_ref[...] = jnp.concatenate([v2, jnp.zeros_like(ones_col)], axis=1)
    s_ref[...] = jnp.dot(h, ws, preferred_element_type=jnp.float32) + bs


def _encode_proj_body(x_ref, few_ref, feb_ref, wq_ref, bq_ref, wk_ref, bk_ref,
                      wv1_ref, wv2_ref, bv_ref, ws_ref, bs_ref,
                      q_ref, k_ref, v1_ref, v2_ref, s_ref):
    x = x_ref[...]
    xa = jnp.dot(x, few_ref[...], preferred_element_type=jnp.float32)
    xa = xa + feb_ref[...]
    h = jnp.concatenate([x, xa], axis=1)
    _proj(h, wq_ref[...], bq_ref[...], wk_ref[...], bk_ref[...],
          wv1_ref[...], wv2_ref[...], bv_ref[...], ws_ref[...], bs_ref[...],
          q_ref, k_ref, v1_ref, v2_ref, s_ref)


def _combine_pool(agg_ref, skip_ref, b_ref, pooled_ref):
    agg_a = agg_ref[0] + agg_ref[1]
    agg_b = agg_ref[2] + agg_ref[3]
    den = agg_a[:, _H:_H + 1] + 1e-16
    num = jnp.concatenate([agg_a[:, :_H], agg_b[:, :_H]], axis=1)
    h = num / den + skip_ref[...]
    g_iota = lax.broadcasted_iota(jnp.int32, (_G, _N), 0)
    onehot = (g_iota == b_ref[...]).astype(jnp.float32)
    cnt = jnp.maximum(jnp.sum(onehot, axis=1, keepdims=True), 1.0)
    pooled_ref[...] = (
        jnp.dot(onehot, h, preferred_element_type=jnp.float32) / cnt)
    return h


def _combine_proj_body(agg_ref, skip_ref, b_ref, wq_ref, bq_ref, wk_ref,
                       bk_ref, wv1_ref, wv2_ref, bv_ref, ws_ref, bs_ref,
                       pooled_ref, q_ref, k_ref, v1_ref, v2_ref, s_ref):
    h = _combine_pool(agg_ref, skip_ref, b_ref, pooled_ref)
    _proj(h, wq_ref[...], bq_ref[...], wk_ref[...], bk_ref[...],
          wv1_ref[...], wv2_ref[...], bv_ref[...], ws_ref[...], bs_ref[...],
          q_ref, k_ref, v1_ref, v2_ref, s_ref)


def _combine_final_body(agg_ref, skip_ref, b_ref, pooled_ref):
    _combine_pool(agg_ref, skip_ref, b_ref, pooled_ref)


_f32 = jnp.float32
_nodes = jax.ShapeDtypeStruct((_N, _D), _f32)
_vhalf = jax.ShapeDtypeStruct((_N, _VW), _f32)
_pooled = jax.ShapeDtypeStruct((_G, _D), _f32)

_encode_proj = pl.pallas_call(
    _encode_proj_body, out_shape=[_nodes, _nodes, _vhalf, _vhalf, _nodes])

_combine_proj = pl.pallas_call(
    _combine_proj_body,
    out_shape=[_pooled, _nodes, _nodes, _vhalf, _vhalf, _nodes])

_combine_final = pl.pallas_call(_combine_final_body, out_shape=[_pooled])


def _zero_acc(zero_hbm, acc_sh, sid):
    pltpu.sync_copy(zero_hbm.at[pl.ds(sid * _RPT, _RPT)],
                    acc_sh.at[pl.ds(sid * _RPT, _RPT)])


def _sc_edge_body(q_hbm, k_hbm, v1_hbm, v2_hbm, src_hbm, dst_hbm, zero_hbm,
                  out_hbm,
                  sidx_all, didx_all, alpha_all, q_bufs, k_bufs, v_bufs,
                  g_sems, s_sems, acc_sh):
    cid = lax.axis_index("c")
    sid = lax.axis_index("s")
    wid = cid * _NS + sid
    lane = lax.iota(jnp.int32, 16)

    extra = jnp.where(wid < _NCHUNK - _NW * (_CPW - 1), 1, 0)
    start = (_CPW - 1) * wid + jnp.minimum(wid, _NCHUNK - _NW * (_CPW - 1))
    cnt = (_CPW - 1) + extra

    pltpu.sync_copy(src_hbm.at[pl.ds(start, _CPW - 1)],
                    sidx_all.at[pl.ds(0, _CPW - 1)])
    pltpu.sync_copy(dst_hbm.at[pl.ds(start, _CPW - 1)],
                    didx_all.at[pl.ds(0, _CPW - 1)])

    @pl.when(extra == 1)
    def _():
        pltpu.sync_copy(src_hbm.at[pl.ds(start + _CPW - 1, 1)],
                        sidx_all.at[pl.ds(_CPW - 1, 1)])
        pltpu.sync_copy(dst_hbm.at[pl.ds(start + _CPW - 1, 1)],
                        didx_all.at[pl.ds(_CPW - 1, 1)])

    _zero_acc(zero_hbm, acc_sh, sid)
    plsc.subcore_barrier()

    def _issue_a(c, b):
        pltpu.async_copy(q_hbm.at[didx_all.at[c]], q_bufs[b], g_sems[b])
        pltpu.async_copy(k_hbm.at[sidx_all.at[c]], k_bufs[b], g_sems[b])
        pltpu.async_copy(v1_hbm.at[sidx_all.at[c]], v_bufs[b], g_sems[b])

    def _wait_gather_a(c, b):
        pltpu.make_async_copy(q_hbm.at[didx_all.at[c]], q_bufs[b],
                              g_sems[b]).wait()
        pltpu.make_async_copy(k_hbm.at[sidx_all.at[c]], k_bufs[b],
                              g_sems[b]).wait()
        pltpu.make_async_copy(v1_hbm.at[sidx_all.at[c]], v_bufs[b],
                              g_sems[b]).wait()

    def _issue_b(c, b):
        pltpu.async_copy(v2_hbm.at[sidx_all.at[c]], v_bufs[b], g_sems[b])

    def _wait_gather_b(c, b):
        pltpu.make_async_copy(v2_hbm.at[sidx_all.at[c]], v_bufs[b],
                              g_sems[b]).wait()

    def _scatter(c, b):
        pltpu.async_copy(v_bufs[b], acc_sh.at[didx_all.at[c]], s_sems[b],
                         add=True)

    def _wait_scatter(c, b):
        pltpu.make_async_copy(v_bufs[b], acc_sh.at[didx_all.at[c]],
                              s_sems[b]).wait()

    _issue_a(0, 0)

    @pl.loop(0, (_CPW + 1) // 2)
    def _pass_a(i):
        for b in range(2):
            c = 2 * i + b
            nb = 1 - b

            @pl.when(c < cnt)
            def _():
                @pl.when(c + 1 < cnt)
                def _():
                    @pl.when(c >= 1)
                    def _():
                        _wait_scatter(c - 1, nb)
                    _issue_a(c + 1, nb)

                _wait_gather_a(c, b)

                for g in range(_C // 16):
                    rows = g * 16 + lane

                    @pl.loop(0, _D, init_carry=jnp.zeros((16,), _f32),
                             unroll=8)
                    def _dot(d, acc):
                        colv = jnp.full((16,), d, jnp.int32)
                        qv = plsc.load_gather(q_bufs[b], [rows, colv])
                        kv = plsc.load_gather(k_bufs[b], [rows, colv])
                        return acc + qv * kv

                    alpha = jnp.exp(_dot * _SCALE)
                    alpha_all[pl.ds(c * _C + g * 16, 16)] = alpha

                    @pl.loop(0, _VW, unroll=8)
                    def _vscale(d):
                        colv = jnp.full((16,), d, jnp.int32)
                        vv = plsc.load_gather(v_bufs[b], [rows, colv])
                        plsc.store_scatter(v_bufs[b], [rows, colv],
                                           vv * alpha)

                _scatter(c, b)

    _wait_scatter(0, 0)
    _wait_scatter(0, 1)

    plsc.subcore_barrier()
    pltpu.sync_copy(acc_sh.at[pl.ds(sid * _RPT, _RPT)],
                    out_hbm.at[cid, pl.ds(sid * _RPT, _RPT)])
    _zero_acc(zero_hbm, acc_sh, sid)
    plsc.subcore_barrier()

    _issue_b(0, 0)

    @pl.loop(0, (_CPW + 1) // 2)
    def _pass_b(i):
        for b in range(2):
            c = 2 * i + b
            nb = 1 - b

            @pl.when(c < cnt)
            def _():
                @pl.when(c + 1 < cnt)
                def _():
                    @pl.when(c >= 1)
                    def _():
                        _wait_scatter(c - 1, nb)
                    _issue_b(c + 1, nb)

                _wait_gather_b(c, b)

                for g in range(_C // 16):
                    rows = g * 16 + lane
                    alpha = alpha_all[pl.ds(c * _C + g * 16, 16)]

                    @pl.loop(0, _VW, unroll=8)
                    def _vscale(d):
                        colv = jnp.full((16,), d, jnp.int32)
                        vv = plsc.load_gather(v_bufs[b], [rows, colv])
                        plsc.store_scatter(v_bufs[b], [rows, colv],
                                           vv * alpha)

                _scatter(c, b)

    _wait_scatter(0, 0)
    _wait_scatter(0, 1)

    plsc.subcore_barrier()
    pltpu.sync_copy(acc_sh.at[pl.ds(sid * _RPT, _RPT)],
                    out_hbm.at[_NC + cid, pl.ds(sid * _RPT, _RPT)])


def _sc_edge(q, k, v1, v2, src, dst, zeros):
    mesh = plsc.VectorSubcoreMesh(core_axis_name="c", subcore_axis_name="s")
    fn = pl.kernel(
        _sc_edge_body,
        out_type=jax.ShapeDtypeStruct((2 * _NC, _N, _VW), _f32),
        mesh=mesh,
        compiler_params=pltpu.CompilerParams(
            use_tc_tiling_on_sc=False, needs_layout_passes=False),
        scratch_types=[
            pltpu.VMEM((_CPW, _C), jnp.int32),
            pltpu.VMEM((_CPW, _C), jnp.int32),
            pltpu.VMEM((_CPW * _C,), _f32),
            [pltpu.VMEM((_C, _D), _f32) for _ in range(2)],
            [pltpu.VMEM((_C, _D), _f32) for _ in range(2)],
            [pltpu.VMEM((_C, _VW), _f32) for _ in range(2)],
            [pltpu.SemaphoreType.DMA for _ in range(2)],
            [pltpu.SemaphoreType.DMA for _ in range(2)],
            pltpu.VMEM_SHARED((_N, _VW), _f32),
        ],
    )
    return fn(q, k, v1, v2, src, dst, zeros)


def kernel(x, edge_index, batch, fe_w, fe_b, wq, bq, wk, bk, wv, bv, ws, bs):
    src = edge_index[0].reshape(_NCHUNK, _C)
    dst = edge_index[1].reshape(_NCHUNK, _C)
    batch_row = batch.reshape(1, _N)
    zeros = jnp.zeros((_N, _VW), _f32)

    q, k, v1, v2, s = _encode_proj(
        x, fe_w, fe_b[None], wq[0], bq[0][None], wk[0], bk[0][None],
        wv[0][:, :_H], wv[0][:, _H:], bv[0][None], ws[0], bs[0][None])

    encs = []
    for l in range(_DEPTH):
        agg = _sc_edge(q, k, v1, v2, src, dst, zeros)
        if l + 1 < _DEPTH:
            pooled, q, k, v1, v2, s = _combine_proj(
                agg, s, batch_row, wq[l + 1], bq[l + 1][None], wk[l + 1],
                bk[l + 1][None], wv[l + 1][:, :_H], wv[l + 1][:, _H:],
                bv[l + 1][None], ws[l + 1], bs[l + 1][None])
        else:
            (pooled,) = _combine_final(agg, s, batch_row)
        encs.append(pooled)
    return jnp.concatenate(encs, axis=-1)


# speedup vs baseline: 7.9116x; 2.6763x over previous
import jax
import jax.numpy as jnp
from jax import lax
from jax.experimental import pallas as pl
from jax.experimental.pallas import tpu as pltpu
from jax.experimental.pallas import tpu_sc as plsc

_N = 10000
_E = 320000
_D = 128
_G = 16
_DEPTH = 4
_H = 64
_VW = 80
_C = 64
_NCHUNK = _E // _C
_NC = 2
_NS = 16
_NW = _NC * _NS
_CPW = (_NCHUNK + _NW - 1) // _NW
_RPT = _N // _NS
_SCALE = 1.0 / float(_D) ** 0.5


def _proj(h, wq, bq, wk, bk, wv1, wv2, bv, ws, bs,
          q_ref, k_ref, v1_ref, v2_ref, s_ref):
    q_ref[...] = jnp.dot(h, wq, preferred_element_type=jnp.float32) + bq
    k_ref[...] = jnp.dot(h, wk, preferred_element_type=jnp.float32) + bk
    v1 = jnp.dot(h, wv1, preferred_element_type=jnp.float32) + bv[:, :_H]
    v2 = jnp.dot(h, wv2, preferred_element_type=jnp.float32) + bv[:, _H:]
    lane = lax.broadcasted_iota(jnp.int32, (_N, _VW - _H), 1)
    ones_col = jnp.where(lane == 0, 1.0, 0.0).astype(jnp.float32)
    v1_ref[...] = jnp.concatenate([v1, ones_col], axis=1)
    v2_ref[...] = jnp.concatenate([v2, jnp.zeros_like(ones_col)], axis=1)
    s_ref[...] = jnp.dot(h, ws, preferred_element_type=jnp.float32) + bs


def _encode_proj_body(x_ref, few_ref, feb_ref, wq_ref, bq_ref, wk_ref, bk_ref,
                      wv1_ref, wv2_ref, bv_ref, ws_ref, bs_ref,
                      q_ref, k_ref, v1_ref, v2_ref, s_ref):
    x = x_ref[...]
    xa = jnp.dot(x, few_ref[...], preferred_element_type=jnp.float32)
    xa = xa + feb_ref[...]
    h = jnp.concatenate([x, xa], axis=1)
    _proj(h, wq_ref[...], bq_ref[...], wk_ref[...], bk_ref[...],
          wv1_ref[...], wv2_ref[...], bv_ref[...], ws_ref[...], bs_ref[...],
          q_ref, k_ref, v1_ref, v2_ref, s_ref)


def _combine_pool(agg_ref, skip_ref, b_ref, pooled_ref):
    agg_a = agg_ref[0] + agg_ref[1]
    agg_b = agg_ref[2] + agg_ref[3]
    den = agg_a[:, _H:_H + 1] + 1e-16
    num = jnp.concatenate([agg_a[:, :_H], agg_b[:, :_H]], axis=1)
    h = num / den + skip_ref[...]
    g_iota = lax.broadcasted_iota(jnp.int32, (_G, _N), 0)
    onehot = (g_iota == b_ref[...]).astype(jnp.float32)
    cnt = jnp.maximum(jnp.sum(onehot, axis=1, keepdims=True), 1.0)
    pooled_ref[...] = (
        jnp.dot(onehot, h, preferred_element_type=jnp.float32) / cnt)
    return h


def _combine_proj_body(agg_ref, skip_ref, b_ref, wq_ref, bq_ref, wk_ref,
                       bk_ref, wv1_ref, wv2_ref, bv_ref, ws_ref, bs_ref,
                       pooled_ref, q_ref, k_ref, v1_ref, v2_ref, s_ref):
    h = _combine_pool(agg_ref, skip_ref, b_ref, pooled_ref)
    _proj(h, wq_ref[...], bq_ref[...], wk_ref[...], bk_ref[...],
          wv1_ref[...], wv2_ref[...], bv_ref[...], ws_ref[...], bs_ref[...],
          q_ref, k_ref, v1_ref, v2_ref, s_ref)


def _combine_final_body(agg_ref, skip_ref, b_ref, pooled_ref):
    _combine_pool(agg_ref, skip_ref, b_ref, pooled_ref)


_f32 = jnp.float32
_nodes = jax.ShapeDtypeStruct((_N, _D), _f32)
_vhalf = jax.ShapeDtypeStruct((_N, _VW), _f32)
_pooled = jax.ShapeDtypeStruct((_G, _D), _f32)

_encode_proj = pl.pallas_call(
    _encode_proj_body, out_shape=[_nodes, _nodes, _vhalf, _vhalf, _nodes])

_combine_proj = pl.pallas_call(
    _combine_proj_body,
    out_shape=[_pooled, _nodes, _nodes, _vhalf, _vhalf, _nodes])

_combine_final = pl.pallas_call(_combine_final_body, out_shape=[_pooled])


def _zero_acc(zero_hbm, acc_sh, sid):
    pltpu.sync_copy(zero_hbm.at[pl.ds(sid * _RPT, _RPT)],
                    acc_sh.at[pl.ds(sid * _RPT, _RPT)])


def _sc_edge_body(q_hbm, k_hbm, v1_hbm, v2_hbm, src_hbm, dst_hbm, zero_hbm,
                  out_hbm,
                  sidx_all, didx_all, alpha_all, q_bufs, k_bufs, v_bufs,
                  g_sems, s_sems, acc_sh):
    cid = lax.axis_index("c")
    sid = lax.axis_index("s")
    wid = cid * _NS + sid
    lane = lax.iota(jnp.int32, 16)

    extra = jnp.where(wid < _NCHUNK - _NW * (_CPW - 1), 1, 0)
    start = (_CPW - 1) * wid + jnp.minimum(wid, _NCHUNK - _NW * (_CPW - 1))
    cnt = (_CPW - 1) + extra

    pltpu.sync_copy(src_hbm.at[pl.ds(start, _CPW - 1)],
                    sidx_all.at[pl.ds(0, _CPW - 1)])
    pltpu.sync_copy(dst_hbm.at[pl.ds(start, _CPW - 1)],
                    didx_all.at[pl.ds(0, _CPW - 1)])

    @pl.when(extra == 1)
    def _():
        pltpu.sync_copy(src_hbm.at[pl.ds(start + _CPW - 1, 1)],
                        sidx_all.at[pl.ds(_CPW - 1, 1)])
        pltpu.sync_copy(dst_hbm.at[pl.ds(start + _CPW - 1, 1)],
                        didx_all.at[pl.ds(_CPW - 1, 1)])

    _zero_acc(zero_hbm, acc_sh, sid)
    plsc.subcore_barrier()

    def _issue_a(c, b):
        pltpu.async_copy(q_hbm.at[didx_all.at[c]], q_bufs[b], g_sems[b])
        pltpu.async_copy(k_hbm.at[sidx_all.at[c]], k_bufs[b], g_sems[b])
        pltpu.async_copy(v1_hbm.at[sidx_all.at[c]], v_bufs[b], g_sems[b])

    def _wait_gather_a(c, b):
        pltpu.make_async_copy(q_hbm.at[didx_all.at[c]], q_bufs[b],
                              g_sems[b]).wait()
        pltpu.make_async_copy(k_hbm.at[sidx_all.at[c]], k_bufs[b],
                              g_sems[b]).wait()
        pltpu.make_async_copy(v1_hbm.at[sidx_all.at[c]], v_bufs[b],
                              g_sems[b]).wait()

    def _issue_b(c, b):
        pltpu.async_copy(v2_hbm.at[sidx_all.at[c]], v_bufs[b], g_sems[b])

    def _wait_gather_b(c, b):
        pltpu.make_async_copy(v2_hbm.at[sidx_all.at[c]], v_bufs[b],
                              g_sems[b]).wait()

    def _scatter(c, b):
        pltpu.async_copy(v_bufs[b], acc_sh.at[didx_all.at[c]], s_sems[b],
                         add=True)

    def _wait_scatter(c, b):
        pltpu.make_async_copy(v_bufs[b], acc_sh.at[didx_all.at[c]],
                              s_sems[b]).wait()

    _issue_a(0, 0)

    @pl.loop(0, (_CPW + 1) // 2)
    def _pass_a(i):
        for b in range(2):
            c = 2 * i + b
            nb = 1 - b

            @pl.when(c < cnt)
            def _():
                @pl.when(c + 1 < cnt)
                def _():
                    @pl.when(c >= 1)
                    def _():
                        _wait_scatter(c - 1, nb)
                    _issue_a(c + 1, nb)

                _wait_gather_a(c, b)

                for g in range(_C // 16):
                    rows = g * 16 + lane

                    @pl.loop(0, _D,
                             init_carry=(jnp.zeros((16,), _f32), lane),
                             unroll=8)
                    def _dot(d, carry):
                        acc, colv = carry
                        qv = plsc.load_gather(q_bufs[b], [rows, colv])
                        kv = plsc.load_gather(k_bufs[b], [rows, colv])
                        nxt = colv + 1
                        nxt = jnp.where(nxt == _D, 0, nxt)
                        return (acc + qv * kv, nxt)

                    alpha = jnp.exp(_dot[0] * _SCALE)
                    alpha_all[pl.ds(c * _C + g * 16, 16)] = alpha

                    @pl.loop(0, _VW, init_carry=lane, unroll=8)
                    def _vscale(d, colv):
                        vv = plsc.load_gather(v_bufs[b], [rows, colv])
                        plsc.store_scatter(v_bufs[b], [rows, colv],
                                           vv * alpha)
                        nxt = colv + 1
                        return jnp.where(nxt == _VW, 0, nxt)

                _scatter(c, b)

    _wait_scatter(0, 0)
    _wait_scatter(0, 1)

    plsc.subcore_barrier()
    pltpu.sync_copy(acc_sh.at[pl.ds(sid * _RPT, _RPT)],
                    out_hbm.at[cid, pl.ds(sid * _RPT, _RPT)])
    _zero_acc(zero_hbm, acc_sh, sid)
    plsc.subcore_barrier()

    _issue_b(0, 0)

    @pl.loop(0, (_CPW + 1) // 2)
    def _pass_b(i):
        for b in range(2):
            c = 2 * i + b
            nb = 1 - b

            @pl.when(c < cnt)
            def _():
                @pl.when(c + 1 < cnt)
                def _():
                    @pl.when(c >= 1)
                    def _():
                        _wait_scatter(c - 1, nb)
                    _issue_b(c + 1, nb)

                _wait_gather_b(c, b)

                for g in range(_C // 16):
                    rows = g * 16 + lane
                    alpha = alpha_all[pl.ds(c * _C + g * 16, 16)]

                    @pl.loop(0, _VW, init_carry=lane, unroll=8)
                    def _vscale(d, colv):
                        vv = plsc.load_gather(v_bufs[b], [rows, colv])
                        plsc.store_scatter(v_bufs[b], [rows, colv],
                                           vv * alpha)
                        nxt = colv + 1
                        return jnp.where(nxt == _VW, 0, nxt)

                _scatter(c, b)

    _wait_scatter(0, 0)
    _wait_scatter(0, 1)

    plsc.subcore_barrier()
    pltpu.sync_copy(acc_sh.at[pl.ds(sid * _RPT, _RPT)],
                    out_hbm.at[_NC + cid, pl.ds(sid * _RPT, _RPT)])


def _sc_edge(q, k, v1, v2, src, dst, zeros):
    mesh = plsc.VectorSubcoreMesh(core_axis_name="c", subcore_axis_name="s")
    fn = pl.kernel(
        _sc_edge_body,
        out_type=jax.ShapeDtypeStruct((2 * _NC, _N, _VW), _f32),
        mesh=mesh,
        compiler_params=pltpu.CompilerParams(
            use_tc_tiling_on_sc=False, needs_layout_passes=False),
        scratch_types=[
            pltpu.VMEM((_CPW, _C), jnp.int32),
            pltpu.VMEM((_CPW, _C), jnp.int32),
            pltpu.VMEM((_CPW * _C,), _f32),
            [pltpu.VMEM((_C, _D), _f32) for _ in range(2)],
            [pltpu.VMEM((_C, _D), _f32) for _ in range(2)],
            [pltpu.VMEM((_C, _VW), _f32) for _ in range(2)],
            [pltpu.SemaphoreType.DMA for _ in range(2)],
            [pltpu.SemaphoreType.DMA for _ in range(2)],
            pltpu.VMEM_SHARED((_N, _VW), _f32),
        ],
    )
    return fn(q, k, v1, v2, src, dst, zeros)


def kernel(x, edge_index, batch, fe_w, fe_b, wq, bq, wk, bk, wv, bv, ws, bs):
    src = edge_index[0].reshape(_NCHUNK, _C)
    dst = edge_index[1].reshape(_NCHUNK, _C)
    batch_row = batch.reshape(1, _N)
    zeros = jnp.zeros((_N, _VW), _f32)

    q, k, v1, v2, s = _encode_proj(
        x, fe_w, fe_b[None], wq[0], bq[0][None], wk[0], bk[0][None],
        wv[0][:, :_H], wv[0][:, _H:], bv[0][None], ws[0], bs[0][None])

    encs = []
    for l in range(_DEPTH):
        agg = _sc_edge(q, k, v1, v2, src, dst, zeros)
        if l + 1 < _DEPTH:
            pooled, q, k, v1, v2, s = _combine_proj(
                agg, s, batch_row, wq[l + 1], bq[l + 1][None], wk[l + 1],
                bk[l + 1][None], wv[l + 1][:, :_H], wv[l + 1][:, _H:],
                bv[l + 1][None], ws[l + 1], bs[l + 1][None])
        else:
            (pooled,) = _combine_final(agg, s, batch_row)
        encs.append(pooled)
    return jnp.concatenate(encs, axis=-1)
